# baseline (speedup 1.0000x reference)
import functools

import jax
import jax.numpy as jnp
from jax import lax
from jax.experimental import pallas as pl
from jax.experimental.pallas import tpu as pltpu
from jax.experimental.pallas import tpu_sc as plsc

B, N, K = 4, 2048, 16
DP, DM = 128, 256
NSEL = 2 * K
TOT = B * N * K

RA = 512
RB = 128
RD = 64


def _proj_body(f_ref, W1_ref, b1_ref, Wq_ref, Wk_ref, Wv_ref,
               q_ref, xk_ref, xv_ref):
    x = jnp.dot(f_ref[...], W1_ref[...], preferred_element_type=jnp.float32)
    x = x + b1_ref[...]
    q_ref[...] = jnp.dot(x, Wq_ref[...], preferred_element_type=jnp.float32)
    xk_ref[...] = jnp.dot(x, Wk_ref[...],
                          preferred_element_type=jnp.float32).astype(jnp.bfloat16)
    xv_ref[...] = jnp.dot(x, Wv_ref[...],
                          preferred_element_type=jnp.float32).astype(jnp.bfloat16)


def _run_proj(ff, W1, b1, Wq, Wk, Wv):
    nb = (B * N) // RA

    def full(shape):
        return pl.BlockSpec(shape, lambda i: tuple(0 for _ in shape))

    return pl.pallas_call(
        _proj_body,
        grid=(nb,),
        in_specs=[
            pl.BlockSpec((RA, DP), lambda i: (i, 0)),
            full((DP, DM)), full((1, DM)), full((DM, DM)), full((DM, DM)),
            full((DM, DM)),
        ],
        out_specs=[pl.BlockSpec((RA, DM), lambda i: (i, 0))] * 3,
        out_shape=[
            jax.ShapeDtypeStruct((B * N, DM), jnp.float32),
            jax.ShapeDtypeStruct((B * N, DM), jnp.bfloat16),
            jax.ShapeDtypeStruct((B * N, DM), jnp.bfloat16),
        ],
    )(ff, W1, b1, Wq, Wk, Wv)


def _topk_body(xyz_ref, xyzT_ref, idx_ref):
    b = pl.program_id(0)
    xi = xyz_ref[...]
    xT = xyzT_ref[0]
    dot = jnp.dot(xi, xT, preferred_element_type=jnp.float32)
    ni = jnp.sum(xi * xi, axis=1, keepdims=True)
    nj = jnp.sum(xT * xT, axis=0, keepdims=True)
    d = (-2.0 * dot + ni) + nj
    col = lax.broadcasted_iota(jnp.int32, (RB, N), 1)
    colk = lax.broadcasted_iota(jnp.int32, (RB, K), 1)

    def body(t, carry):
        d, acc = carry
        m = jnp.min(d, axis=1, keepdims=True)
        sel = jnp.where(d == m, col, N)
        j = jnp.min(sel, axis=1, keepdims=True)
        keep = jnp.logical_and(t % 2 == 0, colk == (t // 2))
        acc = jnp.where(keep, j, acc)
        d = jnp.where(col == j, jnp.inf, d)
        return d, acc

    _, acc = lax.fori_loop(
        0, NSEL, body, (d, jnp.zeros((RB, K), jnp.int32)))
    idx_ref[...] = acc + b * N


def _run_topk(xyzf, xyzT):
    nb = N // RB
    return pl.pallas_call(
        _topk_body,
        grid=(B, nb),
        in_specs=[
            pl.BlockSpec((RB, 3), lambda b, j: (b * nb + j, 0)),
            pl.BlockSpec((1, 3, N), lambda b, j: (b, 0, 0)),
        ],
        out_specs=pl.BlockSpec((RB, K), lambda b, j: (b * nb + j, 0)),
        out_shape=jax.ShapeDtypeStruct((B * N, K), jnp.int32),
    )(xyzf, xyzT)


_NC, _NS = 2, 16
NW = _NC * _NS
BPW = TOT // NW
CH = 128


def _gather3(idx, kt, vt, xt):
    mesh = plsc.VectorSubcoreMesh(core_axis_name="c", subcore_axis_name="s")

    @functools.partial(
        pl.kernel, mesh=mesh,
        out_type=[
            jax.ShapeDtypeStruct((TOT, DP), jnp.float32),
            jax.ShapeDtypeStruct((TOT, DP), jnp.float32),
            jax.ShapeDtypeStruct((TOT, DP), jnp.float32),
        ],
        scratch_types=[
            pltpu.VMEM((CH,), jnp.int32),
            pltpu.VMEM((CH, DP), jnp.float32),
            pltpu.VMEM((CH, DP), jnp.float32),
            pltpu.VMEM((CH, DP), jnp.float32),
            pltpu.SemaphoreType.DMA,
        ],
    )
    def k(idx_hbm, kt_hbm, vt_hbm, xt_hbm, ko_hbm, vo_hbm, xo_hbm,
          idx_v, kb, vb, xb, sem):
        wid = lax.axis_index("s") * _NC + lax.axis_index("c")
        base = wid * BPW

        def step(c, carry):
            off = base + c * CH
            pltpu.sync_copy(idx_hbm.at[pl.ds(off, CH)], idx_v)
            pltpu.async_copy(kt_hbm.at[idx_v], kb, sem).wait()
            pltpu.async_copy(vt_hbm.at[idx_v], vb, sem).wait()
            pltpu.async_copy(xt_hbm.at[idx_v], xb, sem).wait()
            pltpu.sync_copy(kb, ko_hbm.at[pl.ds(off, CH)])
            pltpu.sync_copy(vb, vo_hbm.at[pl.ds(off, CH)])
            pltpu.sync_copy(xb, xo_hbm.at[pl.ds(off, CH)])
            return carry

        lax.fori_loop(0, BPW // CH, step, 0)

    return k(idx, kt, vt, xt)


def _attn_body(q_ref, xyz_ref, f_ref, kg_ref, vg_ref, xg_ref,
               Wd1_ref, bd1_ref, Wd2_ref, bd2_ref, Wg1_ref, bg1_ref,
               Wg2_ref, bg2_ref, W2_ref, b2_ref,
               res_ref, attn_ref):
    xi = xyz_ref[...]
    xg3 = xg_ref[...][:, 0:3].reshape(RD, K, 3)
    delta = (xi[:, None, :] - xg3).reshape(RD * K, 3)
    t = jnp.maximum(
        jnp.dot(delta, Wd1_ref[...], preferred_element_type=jnp.float32)
        + bd1_ref[...], 0.0)
    pos = jnp.dot(t, Wd2_ref[...],
                  preferred_element_type=jnp.float32) + bd2_ref[...]
    pos3 = pos.reshape(RD, K, DM)
    kg3 = kg_ref[...].astype(jnp.float32).reshape(RD, K, DM)
    g3 = q_ref[...][:, None, :] - kg3 + pos3
    h = jnp.maximum(
        jnp.dot(g3.reshape(RD * K, DM), Wg1_ref[...],
                preferred_element_type=jnp.float32) + bg1_ref[...], 0.0)
    h = jnp.dot(h, Wg2_ref[...],
                preferred_element_type=jnp.float32) + bg2_ref[...]
    h3 = h.reshape(RD, K, DM) * (1.0 / 16.0)
    m = jnp.max(h3, axis=1, keepdims=True)
    e = jnp.exp(h3 - m)
    s = jnp.sum(e, axis=1, keepdims=True)
    a3 = e / s
    attn_ref[...] = a3.reshape(RD * K, DM)
    vg3 = vg_ref[...].astype(jnp.float32).reshape(RD, K, DM)
    out = jnp.sum(a3 * (vg3 + pos3), axis=1)
    res_ref[...] = (jnp.dot(out, W2_ref[...],
                            preferred_element_type=jnp.float32)
                    + b2_ref[...] + f_ref[...])


def _run_attn(q, xyzf, ff, kg, vg, xg,
              Wd1, bd1, Wd2, bd2, Wg1, bg1, Wg2, bg2, W2, b2):
    nb = (B * N) // RD

    def full(shape):
        return pl.BlockSpec(shape, lambda i: tuple(0 for _ in shape))

    return pl.pallas_call(
        _attn_body,
        grid=(nb,),
        in_specs=[
            pl.BlockSpec((RD, DM), lambda i: (i, 0)),
            pl.BlockSpec((RD, 3), lambda i: (i, 0)),
            pl.BlockSpec((RD, DP), lambda i: (i, 0)),
            pl.BlockSpec((RD * K, DM), lambda i: (i, 0)),
            pl.BlockSpec((RD * K, DM), lambda i: (i, 0)),
            pl.BlockSpec((RD * K, DP), lambda i: (i, 0)),
            full((3, DM)), full((1, DM)), full((DM, DM)), full((1, DM)),
            full((DM, DM)), full((1, DM)), full((DM, DM)), full((1, DM)),
            full((DM, DP)), full((1, DP)),
        ],
        out_specs=[
            pl.BlockSpec((RD, DP), lambda i: (i, 0)),
            pl.BlockSpec((RD * K, DM), lambda i: (i, 0)),
        ],
        out_shape=[
            jax.ShapeDtypeStruct((B * N, DP), jnp.float32),
            jax.ShapeDtypeStruct((TOT, DM), jnp.float32),
        ],
    )(q, xyzf, ff, kg, vg, xg, Wd1, bd1, Wd2, bd2, Wg1, bg1, Wg2, bg2, W2, b2)


def kernel(xyz, features, W1, b1, W2, b2, Wd1, bd1, Wd2, bd2,
           Wg1, bg1, Wg2, bg2, Wq, Wk, Wv):
    xyzf = xyz.reshape(B * N, 3)
    ff = features.reshape(B * N, DP)
    xyzT = jnp.swapaxes(xyz, 1, 2)

    q, xk, xv = _run_proj(ff, W1, b1.reshape(1, DM), Wq, Wk, Wv)
    idx = _run_topk(xyzf, xyzT).reshape(TOT)
    xt = jnp.concatenate(
        [xyzf, jnp.zeros((B * N, DP - 3), jnp.float32)], axis=1)
    kt = lax.bitcast_convert_type(xk.reshape(B * N, DP, 2), jnp.float32)
    vt = lax.bitcast_convert_type(xv.reshape(B * N, DP, 2), jnp.float32)
    kg, vg, xg = _gather3(idx, kt, vt, xt)
    kgb = lax.bitcast_convert_type(kg[..., None], jnp.bfloat16).reshape(TOT, DM)
    vgb = lax.bitcast_convert_type(vg[..., None], jnp.bfloat16).reshape(TOT, DM)
    res, attn = _run_attn(q, xyzf, ff, kgb, vgb, xg,
                          Wd1, bd1.reshape(1, DM), Wd2, bd2.reshape(1, DM),
                          Wg1, bg1.reshape(1, DM), Wg2, bg2.reshape(1, DM),
                          W2, b2.reshape(1, DP))
    return res.reshape(B, N, DP), attn.reshape(B, N, K, DM)


# speedup vs baseline: 1.5621x; 1.5621x over previous
import functools

import jax
import jax.numpy as jnp
from jax import lax
from jax.experimental import pallas as pl
from jax.experimental.pallas import tpu as pltpu
from jax.experimental.pallas import tpu_sc as plsc

B, N, K = 4, 2048, 16
DP, DM = 128, 256
NSEL = 2 * K
TOT = B * N * K

RA = 512
RB = 128
RD = 64


def _proj_body(f_ref, xyz_ref, W1_ref, b1_ref, Wq_ref, Wk_ref, Wv_ref,
               Wd1_ref, q_ref, xk_ref, xv_ref, p_ref):
    x = jnp.dot(f_ref[...], W1_ref[...], preferred_element_type=jnp.float32)
    x = x + b1_ref[...]
    q_ref[...] = jnp.dot(x, Wq_ref[...], preferred_element_type=jnp.float32)
    xk_ref[...] = jnp.dot(x, Wk_ref[...], preferred_element_type=jnp.float32)
    xv_ref[...] = jnp.dot(x, Wv_ref[...], preferred_element_type=jnp.float32)
    p_ref[...] = jnp.dot(xyz_ref[...], Wd1_ref[...],
                         preferred_element_type=jnp.float32)


def _run_proj(ff, xyzf, W1, b1, Wq, Wk, Wv, Wd1):
    nb = (B * N) // RA

    def full(shape):
        return pl.BlockSpec(shape, lambda i: tuple(0 for _ in shape))

    return pl.pallas_call(
        _proj_body,
        grid=(nb,),
        in_specs=[
            pl.BlockSpec((RA, DP), lambda i: (i, 0)),
            pl.BlockSpec((RA, 3), lambda i: (i, 0)),
            full((DP, DM)), full((1, DM)), full((DM, DM)), full((DM, DM)),
            full((DM, DM)), full((3, DM)),
        ],
        out_specs=[pl.BlockSpec((RA, DM), lambda i: (i, 0))] * 4,
        out_shape=[jax.ShapeDtypeStruct((B * N, DM), jnp.float32)] * 4,
    )(ff, xyzf, W1, b1, Wq, Wk, Wv, Wd1)


def _topk_body(xyz_ref, xyzT_ref, idx_ref):
    b = pl.program_id(0)
    xi = xyz_ref[...]
    xT = xyzT_ref[0]
    dot = jnp.dot(xi, xT, preferred_element_type=jnp.float32)
    ni = jnp.sum(xi * xi, axis=1, keepdims=True)
    nj = jnp.sum(xT * xT, axis=0, keepdims=True)
    d = (-2.0 * dot + ni) + nj
    col = lax.broadcasted_iota(jnp.int32, (RB, N), 1)
    colk = lax.broadcasted_iota(jnp.int32, (RB, K), 1)

    def body(t, carry):
        d, acc = carry
        m = jnp.min(d, axis=1, keepdims=True)
        sel = jnp.where(d == m, col, N)
        j = jnp.min(sel, axis=1, keepdims=True)
        keep = jnp.logical_and(t % 2 == 0, colk == (t // 2))
        acc = jnp.where(keep, j, acc)
        d = jnp.where(col == j, jnp.inf, d)
        return d, acc

    _, acc = lax.fori_loop(
        0, NSEL, body, (d, jnp.zeros((RB, K), jnp.int32)))
    idx_ref[...] = acc + b * N


def _run_topk(xyzf, xyzT):
    nb = N // RB
    return pl.pallas_call(
        _topk_body,
        grid=(B, nb),
        in_specs=[
            pl.BlockSpec((RB, 3), lambda b, j: (b * nb + j, 0)),
            pl.BlockSpec((1, 3, N), lambda b, j: (b, 0, 0)),
        ],
        out_specs=pl.BlockSpec((RB, K), lambda b, j: (b * nb + j, 0)),
        out_shape=jax.ShapeDtypeStruct((B * N, K), jnp.int32),
    )(xyzf, xyzT)


_NC, _NS = 2, 16
NW = _NC * _NS
BPW = TOT // NW
CH = 128


def _gather3(idx, kt, vt, pt):
    mesh = plsc.VectorSubcoreMesh(core_axis_name="c", subcore_axis_name="s")

    @functools.partial(
        pl.kernel, mesh=mesh,
        out_type=[jax.ShapeDtypeStruct((TOT, DM), jnp.float32)] * 3,
        scratch_types=[
            pltpu.VMEM((CH,), jnp.int32),
            pltpu.VMEM((CH, DM), jnp.float32),
            pltpu.VMEM((CH, DM), jnp.float32),
            pltpu.VMEM((CH, DM), jnp.float32),
            pltpu.SemaphoreType.DMA,
        ],
    )
    def k(idx_hbm, kt_hbm, vt_hbm, pt_hbm, ko_hbm, vo_hbm, po_hbm,
          idx_v, kb, vb, pb, sem):
        wid = lax.axis_index("s") * _NC + lax.axis_index("c")
        base = wid * BPW

        def step(c, carry):
            off = base + c * CH
            pltpu.sync_copy(idx_hbm.at[pl.ds(off, CH)], idx_v)
            pltpu.async_copy(kt_hbm.at[idx_v], kb, sem).wait()
            pltpu.async_copy(vt_hbm.at[idx_v], vb, sem).wait()
            pltpu.async_copy(pt_hbm.at[idx_v], pb, sem).wait()
            pltpu.sync_copy(kb, ko_hbm.at[pl.ds(off, CH)])
            pltpu.sync_copy(vb, vo_hbm.at[pl.ds(off, CH)])
            pltpu.sync_copy(pb, po_hbm.at[pl.ds(off, CH)])
            return carry

        lax.fori_loop(0, BPW // CH, step, 0)

    return k(idx, kt, vt, pt)


def _attn_body(q_ref, p_ref, f_ref, kg_ref, vg_ref, pg_ref,
               bd1_ref, Wd2_ref, bd2_ref, Wg1_ref, bg1_ref,
               Wg2_ref, bg2_ref, W2_ref, b2_ref,
               res_ref, attn_ref):
    pi = p_ref[...]
    pg3 = pg_ref[...].reshape(RD, K, DM)
    t = jnp.maximum(pi[:, None, :] - pg3 + bd1_ref[...], 0.0)
    pos = jnp.dot(t.reshape(RD * K, DM), Wd2_ref[...],
                  preferred_element_type=jnp.float32) + bd2_ref[...]
    pos3 = pos.reshape(RD, K, DM)
    kg3 = kg_ref[...].reshape(RD, K, DM)
    g3 = q_ref[...][:, None, :] - kg3 + pos3
    h = jnp.maximum(
        jnp.dot(g3.reshape(RD * K, DM), Wg1_ref[...],
                preferred_element_type=jnp.float32) + bg1_ref[...], 0.0)
    h = jnp.dot(h, Wg2_ref[...],
                preferred_element_type=jnp.float32) + bg2_ref[...]
    h3 = h.reshape(RD, K, DM) * (1.0 / 16.0)
    m = jnp.max(h3, axis=1, keepdims=True)
    e = jnp.exp(h3 - m)
    s = jnp.sum(e, axis=1, keepdims=True)
    a3 = e / s
    attn_ref[...] = a3.reshape(RD * K, DM)
    vg3 = vg_ref[...].reshape(RD, K, DM)
    out = jnp.sum(a3 * (vg3 + pos3), axis=1)
    res_ref[...] = (jnp.dot(out, W2_ref[...],
                            preferred_element_type=jnp.float32)
                    + b2_ref[...] + f_ref[...])


def _run_attn(q, p, ff, kg, vg, pg,
              bd1, Wd2, bd2, Wg1, bg1, Wg2, bg2, W2, b2):
    nb = (B * N) // RD

    def full(shape):
        return pl.BlockSpec(shape, lambda i: tuple(0 for _ in shape))

    return pl.pallas_call(
        _attn_body,
        grid=(nb,),
        in_specs=[
            pl.BlockSpec((RD, DM), lambda i: (i, 0)),
            pl.BlockSpec((RD, DM), lambda i: (i, 0)),
            pl.BlockSpec((RD, DP), lambda i: (i, 0)),
            pl.BlockSpec((RD * K, DM), lambda i: (i, 0)),
            pl.BlockSpec((RD * K, DM), lambda i: (i, 0)),
            pl.BlockSpec((RD * K, DM), lambda i: (i, 0)),
            full((1, DM)), full((DM, DM)), full((1, DM)),
            full((DM, DM)), full((1, DM)), full((DM, DM)), full((1, DM)),
            full((DM, DP)), full((1, DP)),
        ],
        out_specs=[
            pl.BlockSpec((RD, DP), lambda i: (i, 0)),
            pl.BlockSpec((RD * K, DM), lambda i: (i, 0)),
        ],
        out_shape=[
            jax.ShapeDtypeStruct((B * N, DP), jnp.float32),
            jax.ShapeDtypeStruct((TOT, DM), jnp.float32),
        ],
    )(q, p, ff, kg, vg, pg, bd1, Wd2, bd2, Wg1, bg1, Wg2, bg2, W2, b2)


def kernel(xyz, features, W1, b1, W2, b2, Wd1, bd1, Wd2, bd2,
           Wg1, bg1, Wg2, bg2, Wq, Wk, Wv):
    xyzf = xyz.reshape(B * N, 3)
    ff = features.reshape(B * N, DP)
    xyzT = jnp.swapaxes(xyz, 1, 2)

    q, xk, xv, p = _run_proj(ff, xyzf, W1, b1.reshape(1, DM), Wq, Wk, Wv, Wd1)
    idx = _run_topk(xyzf, xyzT).reshape(TOT)
    kg, vg, pg = _gather3(idx, xk, xv, p)
    res, attn = _run_attn(q, p, ff, kg, vg, pg,
                          bd1.reshape(1, DM), Wd2, bd2.reshape(1, DM),
                          Wg1, bg1.reshape(1, DM), Wg2, bg2.reshape(1, DM),
                          W2, b2.reshape(1, DP))
    return res.reshape(B, N, DP), attn.reshape(B, N, K, DM)


# speedup vs baseline: 1.7061x; 1.0921x over previous
import functools

import jax
import jax.numpy as jnp
from jax import lax
from jax.experimental import pallas as pl
from jax.experimental.pallas import tpu as pltpu
from jax.experimental.pallas import tpu_sc as plsc

B, N, K = 4, 2048, 16
DP, DM = 128, 256
NSEL = 2 * K
TOT = B * N * K

RA = 512
RB = 128
RD = 64


def _proj_body(f_ref, xyz_ref, W1_ref, b1_ref, Wq_ref, Wk_ref, Wv_ref,
               Wd1_ref, q_ref, xk_ref, xv_ref, p_ref):
    x = jnp.dot(f_ref[...], W1_ref[...], preferred_element_type=jnp.float32)
    x = x + b1_ref[...]
    q_ref[...] = jnp.dot(x, Wq_ref[...], preferred_element_type=jnp.float32)
    xk_ref[...] = jnp.dot(x, Wk_ref[...], preferred_element_type=jnp.float32)
    xv_ref[...] = jnp.dot(x, Wv_ref[...], preferred_element_type=jnp.float32)
    p_ref[...] = jnp.dot(xyz_ref[...], Wd1_ref[...],
                         preferred_element_type=jnp.float32)


def _run_proj(ff, xyzf, W1, b1, Wq, Wk, Wv, Wd1):
    nb = (B * N) // RA

    def full(shape):
        return pl.BlockSpec(shape, lambda i: tuple(0 for _ in shape))

    return pl.pallas_call(
        _proj_body,
        grid=(nb,),
        in_specs=[
            pl.BlockSpec((RA, DP), lambda i: (i, 0)),
            pl.BlockSpec((RA, 3), lambda i: (i, 0)),
            full((DP, DM)), full((1, DM)), full((DM, DM)), full((DM, DM)),
            full((DM, DM)), full((3, DM)),
        ],
        out_specs=[pl.BlockSpec((RA, DM), lambda i: (i, 0))] * 4,
        out_shape=[jax.ShapeDtypeStruct((B * N, DM), jnp.float32)] * 4,
    )(ff, xyzf, W1, b1, Wq, Wk, Wv, Wd1)


def _topk_body(xyz_ref, xyzT_ref, idx_ref, b0):
    b = pl.program_id(0) + b0
    xi = xyz_ref[...]
    xT = xyzT_ref[0]
    dot = jnp.dot(xi, xT, preferred_element_type=jnp.float32)
    ni = jnp.sum(xi * xi, axis=1, keepdims=True)
    nj = jnp.sum(xT * xT, axis=0, keepdims=True)
    d = (-2.0 * dot + ni) + nj
    col = lax.broadcasted_iota(jnp.int32, (RB, N), 1)
    colk = lax.broadcasted_iota(jnp.int32, (RB, K), 1)

    def body(t, carry):
        d, acc = carry
        m = jnp.min(d, axis=1, keepdims=True)
        sel = jnp.where(d == m, col, N)
        j = jnp.min(sel, axis=1, keepdims=True)
        keep = jnp.logical_and(t % 2 == 0, colk == (t // 2))
        acc = jnp.where(keep, j, acc)
        d = jnp.where(col == j, jnp.inf, d)
        return d, acc

    _, acc = lax.fori_loop(
        0, NSEL, body, (d, jnp.zeros((RB, K), jnp.int32)))
    idx_ref[...] = acc + b * N


def _run_topk(xyzf, xyzT, h, bh):
    nb = N // RB

    def body(xyz_ref, xyzT_ref, idx_ref):
        _topk_body(xyz_ref, xyzT_ref, idx_ref, h * bh)

    return pl.pallas_call(
        body,
        grid=(bh, nb),
        in_specs=[
            pl.BlockSpec((RB, 3), lambda b, j: ((h * bh + b) * nb + j, 0)),
            pl.BlockSpec((1, 3, N), lambda b, j: (h * bh + b, 0, 0)),
        ],
        out_specs=pl.BlockSpec((RB, K), lambda b, j: (b * nb + j, 0)),
        out_shape=jax.ShapeDtypeStruct((bh * N, K), jnp.int32),
    )(xyzf, xyzT)


_NC, _NS = 2, 16
NW = _NC * _NS
BPW = TOT // NW
CH = 128


def _gather3(idx, kt, vt, pt):
    tot = idx.shape[0]
    bpw = tot // NW
    mesh = plsc.VectorSubcoreMesh(core_axis_name="c", subcore_axis_name="s")

    @functools.partial(
        pl.kernel, mesh=mesh,
        out_type=[jax.ShapeDtypeStruct((tot, DM), jnp.float32)] * 3,
        scratch_types=[
            pltpu.VMEM((CH,), jnp.int32),
            pltpu.VMEM((CH, DM), jnp.float32),
            pltpu.VMEM((CH, DM), jnp.float32),
            pltpu.VMEM((CH, DM), jnp.float32),
            pltpu.SemaphoreType.DMA,
        ],
    )
    def k(idx_hbm, kt_hbm, vt_hbm, pt_hbm, ko_hbm, vo_hbm, po_hbm,
          idx_v, kb, vb, pb, sem):
        wid = lax.axis_index("s") * _NC + lax.axis_index("c")
        base = wid * bpw

        def step(c, carry):
            off = base + c * CH
            pltpu.sync_copy(idx_hbm.at[pl.ds(off, CH)], idx_v)
            pltpu.async_copy(kt_hbm.at[idx_v], kb, sem).wait()
            pltpu.async_copy(vt_hbm.at[idx_v], vb, sem).wait()
            pltpu.async_copy(pt_hbm.at[idx_v], pb, sem).wait()
            pltpu.sync_copy(kb, ko_hbm.at[pl.ds(off, CH)])
            pltpu.sync_copy(vb, vo_hbm.at[pl.ds(off, CH)])
            pltpu.sync_copy(pb, po_hbm.at[pl.ds(off, CH)])
            return carry

        lax.fori_loop(0, bpw // CH, step, 0)

    return k(idx, kt, vt, pt)


def _attn_body(q_ref, p_ref, f_ref, kg_ref, vg_ref, pg_ref,
               bd1_ref, Wd2_ref, bd2_ref, Wg1_ref, bg1_ref,
               Wg2_ref, bg2_ref, W2_ref, b2_ref,
               res_ref, attn_ref):
    pi = p_ref[...]
    pg3 = pg_ref[...].reshape(RD, K, DM)
    t = jnp.maximum(pi[:, None, :] - pg3 + bd1_ref[...], 0.0)
    pos = jnp.dot(t.reshape(RD * K, DM), Wd2_ref[...],
                  preferred_element_type=jnp.float32) + bd2_ref[...]
    pos3 = pos.reshape(RD, K, DM)
    kg3 = kg_ref[...].reshape(RD, K, DM)
    g3 = q_ref[...][:, None, :] - kg3 + pos3
    h = jnp.maximum(
        jnp.dot(g3.reshape(RD * K, DM), Wg1_ref[...],
                preferred_element_type=jnp.float32) + bg1_ref[...], 0.0)
    h = jnp.dot(h, Wg2_ref[...],
                preferred_element_type=jnp.float32) + bg2_ref[...]
    h3 = h.reshape(RD, K, DM) * (1.0 / 16.0)
    m = jnp.max(h3, axis=1, keepdims=True)
    e = jnp.exp(h3 - m)
    s = jnp.sum(e, axis=1, keepdims=True)
    a3 = e / s
    attn_ref[...] = a3.reshape(RD * K, DM)
    vg3 = vg_ref[...].reshape(RD, K, DM)
    out = jnp.sum(a3 * (vg3 + pos3), axis=1)
    res_ref[...] = (jnp.dot(out, W2_ref[...],
                            preferred_element_type=jnp.float32)
                    + b2_ref[...] + f_ref[...])


def _run_attn(q, p, ff, kg, vg, pg,
              bd1, Wd2, bd2, Wg1, bg1, Wg2, bg2, W2, b2, h, bh):
    rows = bh * N
    nb = rows // RD
    o = h * nb

    def full(shape):
        return pl.BlockSpec(shape, lambda i: tuple(0 for _ in shape))

    return pl.pallas_call(
        _attn_body,
        grid=(nb,),
        in_specs=[
            pl.BlockSpec((RD, DM), lambda i: (o + i, 0)),
            pl.BlockSpec((RD, DM), lambda i: (o + i, 0)),
            pl.BlockSpec((RD, DP), lambda i: (o + i, 0)),
            pl.BlockSpec((RD * K, DM), lambda i: (i, 0)),
            pl.BlockSpec((RD * K, DM), lambda i: (i, 0)),
            pl.BlockSpec((RD * K, DM), lambda i: (i, 0)),
            full((1, DM)), full((DM, DM)), full((1, DM)),
            full((DM, DM)), full((1, DM)), full((DM, DM)), full((1, DM)),
            full((DM, DP)), full((1, DP)),
        ],
        out_specs=[
            pl.BlockSpec((RD, DP), lambda i: (i, 0)),
            pl.BlockSpec((RD * K, DM), lambda i: (i, 0)),
        ],
        out_shape=[
            jax.ShapeDtypeStruct((rows, DP), jnp.float32),
            jax.ShapeDtypeStruct((rows * K, DM), jnp.float32),
        ],
    )(q, p, ff, kg, vg, pg, bd1, Wd2, bd2, Wg1, bg1, Wg2, bg2, W2, b2)


def kernel(xyz, features, W1, b1, W2, b2, Wd1, bd1, Wd2, bd2,
           Wg1, bg1, Wg2, bg2, Wq, Wk, Wv):
    xyzf = xyz.reshape(B * N, 3)
    ff = features.reshape(B * N, DP)
    xyzT = jnp.swapaxes(xyz, 1, 2)

    q, xk, xv, p = _run_proj(ff, xyzf, W1, b1.reshape(1, DM), Wq, Wk, Wv, Wd1)

    bh = B // 2
    biases = (bd1.reshape(1, DM), Wd2, bd2.reshape(1, DM),
              Wg1, bg1.reshape(1, DM), Wg2, bg2.reshape(1, DM),
              W2, b2.reshape(1, DP))
    idx0 = _run_topk(xyzf, xyzT, 0, bh).reshape(bh * N * K)
    kg0, vg0, pg0 = _gather3(idx0, xk, xv, p)
    idx1 = _run_topk(xyzf, xyzT, 1, bh).reshape(bh * N * K)
    res0, attn0 = _run_attn(q, p, ff, kg0, vg0, pg0, *biases, 0, bh)
    kg1, vg1, pg1 = _gather3(idx1, xk, xv, p)
    res1, attn1 = _run_attn(q, p, ff, kg1, vg1, pg1, *biases, 1, bh)
    res = jnp.concatenate([res0, res1], axis=0)
    attn = jnp.concatenate([attn0, attn1], axis=0)
    return res.reshape(B, N, DP), attn.reshape(B, N, K, DM)


# speedup vs baseline: 1.7775x; 1.0419x over previous
import functools

import jax
import jax.numpy as jnp
from jax import lax
from jax.experimental import pallas as pl
from jax.experimental.pallas import tpu as pltpu
from jax.experimental.pallas import tpu_sc as plsc

B, N, K = 4, 2048, 16
DP, DM = 128, 256
NSEL = 2 * K
TOT = B * N * K

RA = 512
RB = 128
RD = 128


def _proj_body(f_ref, xyz_ref, W1_ref, b1_ref, Wq_ref, Wk_ref, Wv_ref,
               Wd1_ref, q_ref, xk_ref, xv_ref, p_ref):
    x = jnp.dot(f_ref[...], W1_ref[...], preferred_element_type=jnp.float32)
    x = x + b1_ref[...]
    q_ref[...] = jnp.dot(x, Wq_ref[...], preferred_element_type=jnp.float32)
    xk_ref[...] = jnp.dot(x, Wk_ref[...], preferred_element_type=jnp.float32)
    xv_ref[...] = jnp.dot(x, Wv_ref[...], preferred_element_type=jnp.float32)
    p_ref[...] = jnp.dot(xyz_ref[...], Wd1_ref[...],
                         preferred_element_type=jnp.float32)


def _run_proj(ff, xyzf, W1, b1, Wq, Wk, Wv, Wd1):
    nb = (B * N) // RA

    def full(shape):
        return pl.BlockSpec(shape, lambda i: tuple(0 for _ in shape))

    return pl.pallas_call(
        _proj_body,
        grid=(nb,),
        in_specs=[
            pl.BlockSpec((RA, DP), lambda i: (i, 0)),
            pl.BlockSpec((RA, 3), lambda i: (i, 0)),
            full((DP, DM)), full((1, DM)), full((DM, DM)), full((DM, DM)),
            full((DM, DM)), full((3, DM)),
        ],
        out_specs=[pl.BlockSpec((RA, DM), lambda i: (i, 0))] * 4,
        out_shape=[jax.ShapeDtypeStruct((B * N, DM), jnp.float32)] * 4,
    )(ff, xyzf, W1, b1, Wq, Wk, Wv, Wd1)


def _topk_body(xyz_ref, xyzT_ref, idx_ref, b0):
    b = pl.program_id(0) + b0
    xi = xyz_ref[...]
    xT = xyzT_ref[0]
    dot = jnp.dot(xi, xT, preferred_element_type=jnp.float32)
    ni = jnp.sum(xi * xi, axis=1, keepdims=True)
    nj = jnp.sum(xT * xT, axis=0, keepdims=True)
    d = (-2.0 * dot + ni) + nj
    col = lax.broadcasted_iota(jnp.int32, (RB, N), 1)
    colk = lax.broadcasted_iota(jnp.int32, (RB, K), 1)

    def body(t, carry):
        d, acc = carry
        m = jnp.min(d, axis=1, keepdims=True)
        sel = jnp.where(d == m, col, N)
        j = jnp.min(sel, axis=1, keepdims=True)
        keep = jnp.logical_and(t % 2 == 0, colk == (t // 2))
        acc = jnp.where(keep, j, acc)
        d = jnp.where(col == j, jnp.inf, d)
        return d, acc

    _, acc = lax.fori_loop(
        0, NSEL, body, (d, jnp.zeros((RB, K), jnp.int32)))
    idx_ref[...] = acc + b * N


def _run_topk(xyzf, xyzT, h, bh):
    nb = N // RB

    def body(xyz_ref, xyzT_ref, idx_ref):
        _topk_body(xyz_ref, xyzT_ref, idx_ref, h * bh)

    return pl.pallas_call(
        body,
        grid=(bh, nb),
        in_specs=[
            pl.BlockSpec((RB, 3), lambda b, j: ((h * bh + b) * nb + j, 0)),
            pl.BlockSpec((1, 3, N), lambda b, j: (h * bh + b, 0, 0)),
        ],
        out_specs=pl.BlockSpec((RB, K), lambda b, j: (b * nb + j, 0)),
        out_shape=jax.ShapeDtypeStruct((bh * N, K), jnp.int32),
    )(xyzf, xyzT)


_NC, _NS = 2, 16
NW = _NC * _NS
BPW = TOT // NW
CH = 128


def _gather3(idx, kt, vt, pt):
    tot = idx.shape[0]
    bpw = tot // NW
    mesh = plsc.VectorSubcoreMesh(core_axis_name="c", subcore_axis_name="s")

    @functools.partial(
        pl.kernel, mesh=mesh,
        out_type=[jax.ShapeDtypeStruct((tot, DM), jnp.float32)] * 3,
        scratch_types=[
            pltpu.VMEM((CH,), jnp.int32),
            pltpu.VMEM((CH, DM), jnp.float32),
            pltpu.VMEM((CH, DM), jnp.float32),
            pltpu.VMEM((CH, DM), jnp.float32),
            pltpu.SemaphoreType.DMA,
        ],
    )
    def k(idx_hbm, kt_hbm, vt_hbm, pt_hbm, ko_hbm, vo_hbm, po_hbm,
          idx_v, kb, vb, pb, sem):
        wid = lax.axis_index("s") * _NC + lax.axis_index("c")
        base = wid * bpw

        def step(c, carry):
            off = base + c * CH
            pltpu.sync_copy(idx_hbm.at[pl.ds(off, CH)], idx_v)
            pltpu.async_copy(kt_hbm.at[idx_v], kb, sem).wait()
            pltpu.async_copy(vt_hbm.at[idx_v], vb, sem).wait()
            pltpu.async_copy(pt_hbm.at[idx_v], pb, sem).wait()
            pltpu.sync_copy(kb, ko_hbm.at[pl.ds(off, CH)])
            pltpu.sync_copy(vb, vo_hbm.at[pl.ds(off, CH)])
            pltpu.sync_copy(pb, po_hbm.at[pl.ds(off, CH)])
            return carry

        lax.fori_loop(0, bpw // CH, step, 0)

    return k(idx, kt, vt, pt)


def _attn_body(q_ref, p_ref, f_ref, kg_ref, vg_ref, pg_ref,
               bd1_ref, Wd2_ref, bd2_ref, Wg1_ref, bg1_ref,
               Wg2_ref, bg2_ref, W2_ref, b2_ref,
               res_ref, attn_ref):
    pi = p_ref[...]
    pg3 = pg_ref[...].reshape(RD, K, DM)
    t = jnp.maximum(pi[:, None, :] - pg3 + bd1_ref[...], 0.0)
    pos = jnp.dot(t.reshape(RD * K, DM), Wd2_ref[...],
                  preferred_element_type=jnp.float32) + bd2_ref[...]
    pos3 = pos.reshape(RD, K, DM)
    kg3 = kg_ref[...].reshape(RD, K, DM)
    g3 = q_ref[...][:, None, :] - kg3 + pos3
    h = jnp.maximum(
        jnp.dot(g3.reshape(RD * K, DM), Wg1_ref[...],
                preferred_element_type=jnp.float32) + bg1_ref[...], 0.0)
    h = jnp.dot(h, Wg2_ref[...],
                preferred_element_type=jnp.float32) + bg2_ref[...]
    h3 = h.reshape(RD, K, DM) * (1.0 / 16.0)
    m = jnp.max(h3, axis=1, keepdims=True)
    e = jnp.exp(h3 - m)
    s = jnp.sum(e, axis=1, keepdims=True)
    a3 = e / s
    attn_ref[...] = a3.reshape(RD * K, DM)
    vg3 = vg_ref[...].reshape(RD, K, DM)
    out = jnp.sum(a3 * (vg3 + pos3), axis=1)
    res_ref[...] = (jnp.dot(out, W2_ref[...],
                            preferred_element_type=jnp.float32)
                    + b2_ref[...] + f_ref[...])


def _run_attn(q, p, ff, kg, vg, pg,
              bd1, Wd2, bd2, Wg1, bg1, Wg2, bg2, W2, b2, h, bh):
    rows = bh * N
    nb = rows // RD
    o = h * nb

    def full(shape):
        return pl.BlockSpec(shape, lambda i: tuple(0 for _ in shape))

    return pl.pallas_call(
        _attn_body,
        grid=(nb,),
        in_specs=[
            pl.BlockSpec((RD, DM), lambda i: (o + i, 0)),
            pl.BlockSpec((RD, DM), lambda i: (o + i, 0)),
            pl.BlockSpec((RD, DP), lambda i: (o + i, 0)),
            pl.BlockSpec((RD * K, DM), lambda i: (i, 0)),
            pl.BlockSpec((RD * K, DM), lambda i: (i, 0)),
            pl.BlockSpec((RD * K, DM), lambda i: (i, 0)),
            full((1, DM)), full((DM, DM)), full((1, DM)),
            full((DM, DM)), full((1, DM)), full((DM, DM)), full((1, DM)),
            full((DM, DP)), full((1, DP)),
        ],
        out_specs=[
            pl.BlockSpec((RD, DP), lambda i: (i, 0)),
            pl.BlockSpec((RD * K, DM), lambda i: (i, 0)),
        ],
        out_shape=[
            jax.ShapeDtypeStruct((rows, DP), jnp.float32),
            jax.ShapeDtypeStruct((rows * K, DM), jnp.float32),
        ],
    )(q, p, ff, kg, vg, pg, bd1, Wd2, bd2, Wg1, bg1, Wg2, bg2, W2, b2)


def kernel(xyz, features, W1, b1, W2, b2, Wd1, bd1, Wd2, bd2,
           Wg1, bg1, Wg2, bg2, Wq, Wk, Wv):
    xyzf = xyz.reshape(B * N, 3)
    ff = features.reshape(B * N, DP)
    xyzT = jnp.swapaxes(xyz, 1, 2)

    q, xk, xv, p = _run_proj(ff, xyzf, W1, b1.reshape(1, DM), Wq, Wk, Wv, Wd1)

    bh = 1
    biases = (bd1.reshape(1, DM), Wd2, bd2.reshape(1, DM),
              Wg1, bg1.reshape(1, DM), Wg2, bg2.reshape(1, DM),
              W2, b2.reshape(1, DP))
    idxs = [_run_topk(xyzf, xyzT, h, bh).reshape(bh * N * K)
            for h in range(B)]
    gs = [_gather3(i, xk, xv, p) for i in idxs]
    outs = [_run_attn(q, p, ff, *g, *biases, h, bh)
            for h, g in enumerate(gs)]
    res = jnp.concatenate([o[0] for o in outs], axis=0)
    attn = jnp.concatenate([o[1] for o in outs], axis=0)
    return res.reshape(B, N, DP), attn.reshape(B, N, K, DM)


# speedup vs baseline: 1.8275x; 1.0281x over previous
import functools

import jax
import jax.numpy as jnp
from jax import lax
from jax.experimental import pallas as pl
from jax.experimental.pallas import tpu as pltpu
from jax.experimental.pallas import tpu_sc as plsc

B, N, K = 4, 2048, 16
DP, DM = 128, 256
NSEL = 2 * K
TOT = B * N * K

RA = 512
RB = 128
RD = 128


def _proj_body(f_ref, xyz_ref, W1_ref, b1_ref, Wq_ref, Wk_ref, Wv_ref,
               Wd1_ref, q_ref, xk_ref, xv_ref, p_ref):
    x = jnp.dot(f_ref[...], W1_ref[...], preferred_element_type=jnp.float32)
    x = x + b1_ref[...]
    q_ref[...] = jnp.dot(x, Wq_ref[...], preferred_element_type=jnp.float32)
    xk = jnp.dot(x, Wk_ref[...],
                 preferred_element_type=jnp.float32).astype(jnp.bfloat16)
    xv = jnp.dot(x, Wv_ref[...],
                 preferred_element_type=jnp.float32).astype(jnp.bfloat16)
    xk_ref[...] = pltpu.bitcast(xk.reshape(RA * 2, DM // 2), jnp.float32)
    xv_ref[...] = pltpu.bitcast(xv.reshape(RA * 2, DM // 2), jnp.float32)
    p_ref[...] = jnp.dot(xyz_ref[...], Wd1_ref[...],
                         preferred_element_type=jnp.float32)


def _run_proj(ff, xyzf, W1, b1, Wq, Wk, Wv, Wd1):
    nb = (B * N) // RA

    def full(shape):
        return pl.BlockSpec(shape, lambda i: tuple(0 for _ in shape))

    return pl.pallas_call(
        _proj_body,
        grid=(nb,),
        in_specs=[
            pl.BlockSpec((RA, DP), lambda i: (i, 0)),
            pl.BlockSpec((RA, 3), lambda i: (i, 0)),
            full((DP, DM)), full((1, DM)), full((DM, DM)), full((DM, DM)),
            full((DM, DM)), full((3, DM)),
        ],
        out_specs=[
            pl.BlockSpec((RA, DM), lambda i: (i, 0)),
            pl.BlockSpec((RA, DM // 2), lambda i: (i, 0)),
            pl.BlockSpec((RA, DM // 2), lambda i: (i, 0)),
            pl.BlockSpec((RA, DM), lambda i: (i, 0)),
        ],
        out_shape=[
            jax.ShapeDtypeStruct((B * N, DM), jnp.float32),
            jax.ShapeDtypeStruct((B * N, DM // 2), jnp.float32),
            jax.ShapeDtypeStruct((B * N, DM // 2), jnp.float32),
            jax.ShapeDtypeStruct((B * N, DM), jnp.float32),
        ],
    )(ff, xyzf, W1, b1, Wq, Wk, Wv, Wd1)


def _topk_body(xyz_ref, xyzT_ref, idx_ref, b0):
    b = pl.program_id(0) + b0
    xi = xyz_ref[...]
    xT = xyzT_ref[0]
    dot = jnp.dot(xi, xT, preferred_element_type=jnp.float32)
    ni = jnp.sum(xi * xi, axis=1, keepdims=True)
    nj = jnp.sum(xT * xT, axis=0, keepdims=True)
    d = (-2.0 * dot + ni) + nj
    col = lax.broadcasted_iota(jnp.int32, (RB, N), 1)
    colk = lax.broadcasted_iota(jnp.int32, (RB, K), 1)

    def body(t, carry):
        d, acc = carry
        m = jnp.min(d, axis=1, keepdims=True)
        sel = jnp.where(d == m, col, N)
        j = jnp.min(sel, axis=1, keepdims=True)
        keep = jnp.logical_and(t % 2 == 0, colk == (t // 2))
        acc = jnp.where(keep, j, acc)
        d = jnp.where(col == j, jnp.inf, d)
        return d, acc

    _, acc = lax.fori_loop(
        0, NSEL, body, (d, jnp.zeros((RB, K), jnp.int32)))
    idx_ref[...] = acc + b * N


def _run_topk(xyzf, xyzT, h, bh):
    nb = N // RB

    def body(xyz_ref, xyzT_ref, idx_ref):
        _topk_body(xyz_ref, xyzT_ref, idx_ref, h * bh)

    return pl.pallas_call(
        body,
        grid=(bh, nb),
        in_specs=[
            pl.BlockSpec((RB, 3), lambda b, j: ((h * bh + b) * nb + j, 0)),
            pl.BlockSpec((1, 3, N), lambda b, j: (h * bh + b, 0, 0)),
        ],
        out_specs=pl.BlockSpec((RB, K), lambda b, j: (b * nb + j, 0)),
        out_shape=jax.ShapeDtypeStruct((bh * N, K), jnp.int32),
    )(xyzf, xyzT)


_NC, _NS = 2, 16
NW = _NC * _NS
BPW = TOT // NW
CH = 128


def _gather3(idx, kt, vt, pt):
    tot = idx.shape[0]
    bpw = tot // NW
    mesh = plsc.VectorSubcoreMesh(core_axis_name="c", subcore_axis_name="s")

    @functools.partial(
        pl.kernel, mesh=mesh,
        out_type=[
            jax.ShapeDtypeStruct((tot, DM // 2), jnp.float32),
            jax.ShapeDtypeStruct((tot, DM // 2), jnp.float32),
            jax.ShapeDtypeStruct((tot, DM), jnp.float32),
        ],
        scratch_types=[
            pltpu.VMEM((CH,), jnp.int32),
            pltpu.VMEM((CH, DM // 2), jnp.float32),
            pltpu.VMEM((CH, DM // 2), jnp.float32),
            pltpu.VMEM((CH, DM), jnp.float32),
            pltpu.SemaphoreType.DMA,
        ],
    )
    def k(idx_hbm, kt_hbm, vt_hbm, pt_hbm, ko_hbm, vo_hbm, po_hbm,
          idx_v, kb, vb, pb, sem):
        wid = lax.axis_index("s") * _NC + lax.axis_index("c")
        base = wid * bpw

        def step(c, carry):
            off = base + c * CH
            pltpu.sync_copy(idx_hbm.at[pl.ds(off, CH)], idx_v)
            pltpu.async_copy(kt_hbm.at[idx_v], kb, sem).wait()
            pltpu.async_copy(vt_hbm.at[idx_v], vb, sem).wait()
            pltpu.async_copy(pt_hbm.at[idx_v], pb, sem).wait()
            pltpu.sync_copy(kb, ko_hbm.at[pl.ds(off, CH)])
            pltpu.sync_copy(vb, vo_hbm.at[pl.ds(off, CH)])
            pltpu.sync_copy(pb, po_hbm.at[pl.ds(off, CH)])
            return carry

        lax.fori_loop(0, bpw // CH, step, 0)

    return k(idx, kt, vt, pt)


def _attn_body(q_ref, p_ref, f_ref, kg_ref, vg_ref, pg_ref,
               bd1_ref, Wd2_ref, bd2_ref, Wg1_ref, bg1_ref,
               Wg2_ref, bg2_ref, W2_ref, b2_ref,
               res_ref, attn_ref):
    pi = p_ref[...]
    pg3 = pg_ref[...].reshape(RD, K, DM)
    t = jnp.maximum(pi[:, None, :] - pg3 + bd1_ref[...], 0.0)
    pos = jnp.dot(t.reshape(RD * K, DM), Wd2_ref[...],
                  preferred_element_type=jnp.float32) + bd2_ref[...]
    pos3 = pos.reshape(RD, K, DM)
    kg3 = (pltpu.bitcast(kg_ref[...], jnp.bfloat16)
           .reshape(RD, K, DM).astype(jnp.float32))
    g3 = q_ref[...][:, None, :] - kg3 + pos3
    h = jnp.maximum(
        jnp.dot(g3.reshape(RD * K, DM), Wg1_ref[...],
                preferred_element_type=jnp.float32) + bg1_ref[...], 0.0)
    h = jnp.dot(h, Wg2_ref[...],
                preferred_element_type=jnp.float32) + bg2_ref[...]
    h3 = h.reshape(RD, K, DM) * (1.0 / 16.0)
    m = jnp.max(h3, axis=1, keepdims=True)
    e = jnp.exp(h3 - m)
    s = jnp.sum(e, axis=1, keepdims=True)
    a3 = e / s
    attn_ref[...] = a3.reshape(RD * K, DM)
    vg3 = (pltpu.bitcast(vg_ref[...], jnp.bfloat16)
           .reshape(RD, K, DM).astype(jnp.float32))
    out = jnp.sum(a3 * (vg3 + pos3), axis=1)
    res_ref[...] = (jnp.dot(out, W2_ref[...],
                            preferred_element_type=jnp.float32)
                    + b2_ref[...] + f_ref[...])


def _run_attn(q, p, ff, kg, vg, pg,
              bd1, Wd2, bd2, Wg1, bg1, Wg2, bg2, W2, b2, h, bh):
    rows = bh * N
    nb = rows // RD
    o = h * nb

    def full(shape):
        return pl.BlockSpec(shape, lambda i: tuple(0 for _ in shape))

    return pl.pallas_call(
        _attn_body,
        grid=(nb,),
        in_specs=[
            pl.BlockSpec((RD, DM), lambda i: (o + i, 0)),
            pl.BlockSpec((RD, DM), lambda i: (o + i, 0)),
            pl.BlockSpec((RD, DP), lambda i: (o + i, 0)),
            pl.BlockSpec((RD * K, DM // 2), lambda i: (i, 0)),
            pl.BlockSpec((RD * K, DM // 2), lambda i: (i, 0)),
            pl.BlockSpec((RD * K, DM), lambda i: (i, 0)),
            full((1, DM)), full((DM, DM)), full((1, DM)),
            full((DM, DM)), full((1, DM)), full((DM, DM)), full((1, DM)),
            full((DM, DP)), full((1, DP)),
        ],
        out_specs=[
            pl.BlockSpec((RD, DP), lambda i: (i, 0)),
            pl.BlockSpec((RD * K, DM), lambda i: (i, 0)),
        ],
        out_shape=[
            jax.ShapeDtypeStruct((rows, DP), jnp.float32),
            jax.ShapeDtypeStruct((rows * K, DM), jnp.float32),
        ],
    )(q, p, ff, kg, vg, pg, bd1, Wd2, bd2, Wg1, bg1, Wg2, bg2, W2, b2)


def kernel(xyz, features, W1, b1, W2, b2, Wd1, bd1, Wd2, bd2,
           Wg1, bg1, Wg2, bg2, Wq, Wk, Wv):
    xyzf = xyz.reshape(B * N, 3)
    ff = features.reshape(B * N, DP)
    xyzT = jnp.swapaxes(xyz, 1, 2)

    q, xk, xv, p = _run_proj(ff, xyzf, W1, b1.reshape(1, DM), Wq, Wk, Wv, Wd1)

    bh = 1
    biases = (bd1.reshape(1, DM), Wd2, bd2.reshape(1, DM),
              Wg1, bg1.reshape(1, DM), Wg2, bg2.reshape(1, DM),
              W2, b2.reshape(1, DP))
    idxs = [_run_topk(xyzf, xyzT, h, bh).reshape(bh * N * K)
            for h in range(B)]
    gs = [_gather3(i, xk, xv, p) for i in idxs]
    outs = [_run_attn(q, p, ff, *g, *biases, h, bh)
            for h, g in enumerate(gs)]
    res = jnp.concatenate([o[0] for o in outs], axis=0)
    attn = jnp.concatenate([o[1] for o in outs], axis=0)
    return res.reshape(B, N, DP), attn.reshape(B, N, K, DM)


# speedup vs baseline: 1.9962x; 1.0924x over previous
import functools

import jax
import jax.numpy as jnp
from jax import lax
from jax.experimental import pallas as pl
from jax.experimental.pallas import tpu as pltpu
from jax.experimental.pallas import tpu_sc as plsc

B, N, K = 4, 2048, 16
DP, DM = 128, 256
NSEL = 2 * K
TOT = B * N * K

RA = 512
RB = 128
RD = 128


def _proj_body(f_ref, xyz_ref, W1_ref, b1_ref, Wq_ref, Wk_ref, Wv_ref,
               Wd1_ref, q_ref, xk_ref, xv_ref, p_ref):
    x = jnp.dot(f_ref[...], W1_ref[...], preferred_element_type=jnp.float32)
    x = x + b1_ref[...]
    q_ref[...] = jnp.dot(x, Wq_ref[...], preferred_element_type=jnp.float32)
    xk = jnp.dot(x, Wk_ref[...],
                 preferred_element_type=jnp.float32).astype(jnp.bfloat16)
    xv = jnp.dot(x, Wv_ref[...],
                 preferred_element_type=jnp.float32).astype(jnp.bfloat16)
    xk_ref[...] = pltpu.bitcast(xk.reshape(RA * 2, DM // 2), jnp.float32)
    xv_ref[...] = pltpu.bitcast(xv.reshape(RA * 2, DM // 2), jnp.float32)
    pp = jnp.dot(xyz_ref[...], Wd1_ref[...],
                 preferred_element_type=jnp.float32).astype(jnp.bfloat16)
    p_ref[...] = pltpu.bitcast(pp.reshape(RA * 2, DM // 2), jnp.float32)


def _run_proj(ff, xyzf, W1, b1, Wq, Wk, Wv, Wd1):
    nb = (B * N) // RA

    def full(shape):
        return pl.BlockSpec(shape, lambda i: tuple(0 for _ in shape))

    return pl.pallas_call(
        _proj_body,
        grid=(nb,),
        in_specs=[
            pl.BlockSpec((RA, DP), lambda i: (i, 0)),
            pl.BlockSpec((RA, 3), lambda i: (i, 0)),
            full((DP, DM)), full((1, DM)), full((DM, DM)), full((DM, DM)),
            full((DM, DM)), full((3, DM)),
        ],
        out_specs=[
            pl.BlockSpec((RA, DM), lambda i: (i, 0)),
            pl.BlockSpec((RA, DM // 2), lambda i: (i, 0)),
            pl.BlockSpec((RA, DM // 2), lambda i: (i, 0)),
            pl.BlockSpec((RA, DM // 2), lambda i: (i, 0)),
        ],
        out_shape=[
            jax.ShapeDtypeStruct((B * N, DM), jnp.float32),
            jax.ShapeDtypeStruct((B * N, DM // 2), jnp.float32),
            jax.ShapeDtypeStruct((B * N, DM // 2), jnp.float32),
            jax.ShapeDtypeStruct((B * N, DM // 2), jnp.float32),
        ],
    )(ff, xyzf, W1, b1, Wq, Wk, Wv, Wd1)


def _topk_body(xyz_ref, xyzT_ref, idx_ref, b0):
    b = pl.program_id(0) + b0
    xi = xyz_ref[...]
    xT = xyzT_ref[0]
    dot = jnp.dot(xi, xT, preferred_element_type=jnp.float32)
    ni = jnp.sum(xi * xi, axis=1, keepdims=True)
    nj = jnp.sum(xT * xT, axis=0, keepdims=True)
    d = (-2.0 * dot + ni) + nj
    col = lax.broadcasted_iota(jnp.int32, (RB, N), 1)
    colk = lax.broadcasted_iota(jnp.int32, (RB, K), 1)

    def body(t, carry):
        d, acc = carry
        m = jnp.min(d, axis=1, keepdims=True)
        sel = jnp.where(d == m, col, N)
        j = jnp.min(sel, axis=1, keepdims=True)
        keep = jnp.logical_and(t % 2 == 0, colk == (t // 2))
        acc = jnp.where(keep, j, acc)
        d = jnp.where(col == j, jnp.inf, d)
        return d, acc

    _, acc = lax.fori_loop(
        0, NSEL, body, (d, jnp.zeros((RB, K), jnp.int32)))
    idx_ref[...] = acc + b * N


def _run_topk(xyzf, xyzT, h, bh):
    nb = N // RB

    def body(xyz_ref, xyzT_ref, idx_ref):
        _topk_body(xyz_ref, xyzT_ref, idx_ref, h * bh)

    return pl.pallas_call(
        body,
        grid=(bh, nb),
        in_specs=[
            pl.BlockSpec((RB, 3), lambda b, j: ((h * bh + b) * nb + j, 0)),
            pl.BlockSpec((1, 3, N), lambda b, j: (h * bh + b, 0, 0)),
        ],
        out_specs=pl.BlockSpec((RB, K), lambda b, j: (b * nb + j, 0)),
        out_shape=jax.ShapeDtypeStruct((bh * N, K), jnp.int32),
    )(xyzf, xyzT)


_NC, _NS = 2, 16
NW = _NC * _NS
BPW = TOT // NW
CH = 128


def _gather3(idx, kt, vt, pt):
    tot = idx.shape[0]
    bpw = tot // NW
    mesh = plsc.VectorSubcoreMesh(core_axis_name="c", subcore_axis_name="s")

    @functools.partial(
        pl.kernel, mesh=mesh,
        out_type=[jax.ShapeDtypeStruct((tot, DM // 2), jnp.float32)] * 3,
        scratch_types=[
            pltpu.VMEM((CH,), jnp.int32),
            pltpu.VMEM((CH, DM // 2), jnp.float32),
            pltpu.VMEM((CH, DM // 2), jnp.float32),
            pltpu.VMEM((CH, DM // 2), jnp.float32),
            pltpu.SemaphoreType.DMA,
        ],
    )
    def k(idx_hbm, kt_hbm, vt_hbm, pt_hbm, ko_hbm, vo_hbm, po_hbm,
          idx_v, kb, vb, pb, sem):
        wid = lax.axis_index("s") * _NC + lax.axis_index("c")
        base = wid * bpw

        def step(c, carry):
            off = base + c * CH
            pltpu.sync_copy(idx_hbm.at[pl.ds(off, CH)], idx_v)
            pltpu.async_copy(kt_hbm.at[idx_v], kb, sem).wait()
            pltpu.async_copy(vt_hbm.at[idx_v], vb, sem).wait()
            pltpu.async_copy(pt_hbm.at[idx_v], pb, sem).wait()
            pltpu.sync_copy(kb, ko_hbm.at[pl.ds(off, CH)])
            pltpu.sync_copy(vb, vo_hbm.at[pl.ds(off, CH)])
            pltpu.sync_copy(pb, po_hbm.at[pl.ds(off, CH)])
            return carry

        lax.fori_loop(0, bpw // CH, step, 0)

    return k(idx, kt, vt, pt)


def _attn_body(q_ref, p_ref, f_ref, kg_ref, vg_ref, pg_ref,
               bd1_ref, Wd2_ref, bd2_ref, Wg1_ref, bg1_ref,
               Wg2_ref, bg2_ref, W2_ref, b2_ref,
               res_ref, attn_ref):
    pi = (pltpu.bitcast(p_ref[...], jnp.bfloat16)
          .reshape(RD, DM).astype(jnp.float32))
    pg3 = (pltpu.bitcast(pg_ref[...], jnp.bfloat16)
           .reshape(RD, K, DM).astype(jnp.float32))
    t = jnp.maximum(pi[:, None, :] - pg3 + bd1_ref[...], 0.0)
    pos = jnp.dot(t.reshape(RD * K, DM), Wd2_ref[...],
                  preferred_element_type=jnp.float32) + bd2_ref[...]
    pos3 = pos.reshape(RD, K, DM)
    kg3 = (pltpu.bitcast(kg_ref[...], jnp.bfloat16)
           .reshape(RD, K, DM).astype(jnp.float32))
    g3 = q_ref[...][:, None, :] - kg3 + pos3
    h = jnp.maximum(
        jnp.dot(g3.reshape(RD * K, DM), Wg1_ref[...],
                preferred_element_type=jnp.float32) + bg1_ref[...], 0.0)
    h = jnp.dot(h, Wg2_ref[...],
                preferred_element_type=jnp.float32) + bg2_ref[...]
    h3 = h.reshape(RD, K, DM) * (1.0 / 16.0)
    m = jnp.max(h3, axis=1, keepdims=True)
    e = jnp.exp(h3 - m)
    s = jnp.sum(e, axis=1, keepdims=True)
    a3 = e / s
    attn_ref[...] = a3.reshape(RD * K, DM)
    vg3 = (pltpu.bitcast(vg_ref[...], jnp.bfloat16)
           .reshape(RD, K, DM).astype(jnp.float32))
    out = jnp.sum(a3 * (vg3 + pos3), axis=1)
    res_ref[...] = (jnp.dot(out, W2_ref[...],
                            preferred_element_type=jnp.float32)
                    + b2_ref[...] + f_ref[...])


def _run_attn(q, p, ff, kg, vg, pg,
              bd1, Wd2, bd2, Wg1, bg1, Wg2, bg2, W2, b2, h, bh,
              res_in, attn_in):
    rows = bh * N
    nb = rows // RD
    o = h * nb

    def full(shape):
        return pl.BlockSpec(shape, lambda i: tuple(0 for _ in shape))

    in_specs = [
        pl.BlockSpec((RD, DM), lambda i: (o + i, 0)),
        pl.BlockSpec((RD, DM // 2), lambda i: (o + i, 0)),
        pl.BlockSpec((RD, DP), lambda i: (o + i, 0)),
        pl.BlockSpec((RD * K, DM // 2), lambda i: (i, 0)),
        pl.BlockSpec((RD * K, DM // 2), lambda i: (i, 0)),
        pl.BlockSpec((RD * K, DM // 2), lambda i: (i, 0)),
        full((1, DM)), full((DM, DM)), full((1, DM)),
        full((DM, DM)), full((1, DM)), full((DM, DM)), full((1, DM)),
        full((DM, DP)), full((1, DP)),
    ]
    args = [q, p, ff, kg, vg, pg, bd1, Wd2, bd2, Wg1, bg1, Wg2, bg2, W2, b2]
    body = _attn_body
    aliases = {}
    if h > 0:
        in_specs = in_specs + [pl.BlockSpec(memory_space=pl.ANY)] * 2
        args = args + [res_in, attn_in]
        aliases = {15: 0, 16: 1}
        body = lambda *rs: _attn_body(*rs[:15], rs[17], rs[18])

    return pl.pallas_call(
        body,
        grid=(nb,),
        in_specs=in_specs,
        out_specs=[
            pl.BlockSpec((RD, DP), lambda i: (o + i, 0)),
            pl.BlockSpec((RD * K, DM), lambda i: (o + i, 0)),
        ],
        out_shape=[
            jax.ShapeDtypeStruct((B * N, DP), jnp.float32),
            jax.ShapeDtypeStruct((TOT, DM), jnp.float32),
        ],
        input_output_aliases=aliases,
    )(*args)


def kernel(xyz, features, W1, b1, W2, b2, Wd1, bd1, Wd2, bd2,
           Wg1, bg1, Wg2, bg2, Wq, Wk, Wv):
    xyzf = xyz.reshape(B * N, 3)
    ff = features.reshape(B * N, DP)
    xyzT = jnp.swapaxes(xyz, 1, 2)

    q, xk, xv, p = _run_proj(ff, xyzf, W1, b1.reshape(1, DM), Wq, Wk, Wv, Wd1)

    bh = 1
    biases = (bd1.reshape(1, DM), Wd2, bd2.reshape(1, DM),
              Wg1, bg1.reshape(1, DM), Wg2, bg2.reshape(1, DM),
              W2, b2.reshape(1, DP))
    idxs = [_run_topk(xyzf, xyzT, h, bh).reshape(bh * N * K)
            for h in range(B)]
    gs = [_gather3(i, xk, xv, p) for i in idxs]
    res, attn = None, None
    for h, g in enumerate(gs):
        res, attn = _run_attn(q, p, ff, *g, *biases, h, bh, res, attn)
    return res.reshape(B, N, DP), attn.reshape(B, N, K, DM)
